# baseline (speedup 1.0000x reference)
import jax
import jax.numpy as jnp
from jax import lax
from jax.experimental import pallas as pl
from jax.experimental.pallas import tpu as pltpu
from jax.experimental.pallas import tpu_sc as plsc

_V = 100000
_B = 4096
_C = 200
_D = 64
_CP = 208
_NW = 32
_NSUB = 16
_RPW = _B // _NW
_NCH = _CP // 16
_HALF = _CP // 2
_NBUF = 4
_NIN = 8
_NPH = 2
_DC = _D // _NPH
_PK = _DC // 2
_VPT = _V // _NSUB
_RB = 64
_NBLK = _RPW // _RB


def _sc_body(x_hbm, cand_hbm, table_hbm, out_hbm, idx_r, x_r, rows_v, out_v,
             chunk_s, *sems):
    gsems = sems[:_NBUF]
    csems = sems[_NBUF:]
    wid = lax.axis_index("s") * 2 + lax.axis_index("c")
    sid = lax.axis_index("s")
    base = wid * _RPW

    lane = lax.broadcasted_iota(jnp.int32, (16,), 0)
    himask = jnp.full((16,), -65536, jnp.int32)

    def blk_body(blk, carry):
        rbase = base + blk * _RB
        for p in range(_NPH):
            pltpu.sync_copy(table_hbm.at[p, pl.ds(sid * _VPT, _VPT)],
                            chunk_s.at[pl.ds(sid * _VPT, _VPT)])
            plsc.subcore_barrier()

            def copy_descs(row, s):
                return (
                    pltpu.make_async_copy(cand_hbm.at[rbase + row],
                                          idx_r.at[s], csems[s]),
                    pltpu.make_async_copy(
                        x_hbm.at[rbase + row, pl.ds(p * _DC, _DC)],
                        x_r.at[s], csems[s]),
                )

            def gather_descs(s, b):
                return (
                    pltpu.make_async_copy(chunk_s.at[idx_r.at[s, 0]],
                                          rows_v.at[b, pl.ds(0, _HALF)],
                                          gsems[b]),
                    pltpu.make_async_copy(chunk_s.at[idx_r.at[s, 1]],
                                          rows_v.at[b, pl.ds(_HALF, _HALF)],
                                          gsems[b]),
                )

            def issue(descs):
                for desc in descs:
                    desc.start()

            def compute(row, s, b):
                bb = jnp.full((16,), b, jnp.int32)
                ss = jnp.full((16,), s, jnp.int32)

                def d_body(k, accs):
                    kk = jnp.full((16,), k, jnp.int32)
                    xb0 = plsc.load_gather(x_r, [ss, kk * 2])
                    xb1 = plsc.load_gather(x_r, [ss, kk * 2 + 1])
                    new = []
                    for j in range(_NCH):
                        w = plsc.load_gather(rows_v,
                                             [bb, lane + (16 * j), kk])
                        f0 = plsc.bitcast(w << 16, jnp.float32)
                        f1 = plsc.bitcast(w & himask, jnp.float32)
                        new.append(accs[j] + xb0 * f0 + xb1 * f1)
                    return tuple(new)

                accs = lax.fori_loop(
                    0, _PK, d_body,
                    tuple(jnp.zeros((16,), jnp.float32)
                          for _ in range(_NCH)))
                for j in range(_NCH):
                    if p == 0:
                        out_v[row, pl.ds(16 * j, 16)] = accs[j]
                    else:
                        out_v[row, pl.ds(16 * j, 16)] = (
                            out_v[row, pl.ds(16 * j, 16)] + accs[j])

            for r in range(_NIN):
                issue(copy_descs(r, r))
            for r in range(_NBUF - 1):
                for desc in copy_descs(r, r):
                    desc.wait()
                issue(gather_descs(r, r))

            def outer_body(r2, carry2):
                for i in range(_NIN):
                    row = r2 * _NIN + i
                    b = i % _NBUF
                    nxt = row + (_NBUF - 1)
                    s3 = (i + _NBUF - 1) % _NIN

                    for desc in gather_descs(i, b):
                        desc.wait()

                    @pl.when(nxt < _RB)
                    def _():
                        for desc in copy_descs(nxt, s3):
                            desc.wait()
                        issue(gather_descs(s3, (b + _NBUF - 1) % _NBUF))

                    compute(row, i, b)

                    @pl.when(row + _NIN < _RB)
                    def _():
                        issue(copy_descs(row + _NIN, i))
                return carry2

            lax.fori_loop(0, _RB // _NIN, outer_body, 0)
            plsc.subcore_barrier()

        pltpu.sync_copy(out_v, out_hbm.at[pl.ds(rbase, _RB)])
        return carry

    lax.fori_loop(0, _NBLK, blk_body, 0)


def kernel(x, candidates, table):
    cand = candidates.astype(jnp.int32)
    cand = jnp.concatenate(
        [cand, jnp.zeros((_B, _CP - _C), jnp.int32)], axis=1)
    cand = cand.reshape(_B, 2, _HALF)

    tu = jax.lax.bitcast_convert_type(
        table.astype(jnp.bfloat16), jnp.uint16)
    w = tu[:, 0::2].astype(jnp.uint32) | (
        tu[:, 1::2].astype(jnp.uint32) << 16)
    table_t = jax.lax.bitcast_convert_type(
        w, jnp.int32).reshape(_V, _NPH, _PK).transpose(1, 0, 2)

    mesh = plsc.VectorSubcoreMesh(core_axis_name="c", subcore_axis_name="s")
    out = pl.kernel(
        _sc_body,
        mesh=mesh,
        compiler_params=pltpu.CompilerParams(
            needs_layout_passes=False, use_tc_tiling_on_sc=False),
        out_type=jax.ShapeDtypeStruct((_B, _CP), jnp.float32),
        scratch_types=[
            pltpu.VMEM((_NIN, 2, _HALF), jnp.int32),
            pltpu.VMEM((_NIN, _DC), jnp.float32),
            pltpu.VMEM((_NBUF, _CP, _PK), jnp.int32),
            pltpu.VMEM((_RB, _CP), jnp.float32),
            pltpu.MemorySpace.VMEM_SHARED((_V, _PK), jnp.int32),
        ] + [pltpu.SemaphoreType.DMA] * (_NBUF + _NIN),
    )(x, cand, table_t)
    return out[:, :_C]


# speedup vs baseline: 1.1232x; 1.1232x over previous
import jax
import jax.numpy as jnp
from jax import lax
from jax.experimental import pallas as pl
from jax.experimental.pallas import tpu as pltpu
from jax.experimental.pallas import tpu_sc as plsc

_V = 100000
_B = 4096
_C = 200
_D = 64
_CP = 208
_NW = 32
_NSUB = 16
_RPW = _B // _NW
_NCH = _CP // 16
_GH = _C // 2
_NBUF = 8
_NPH = 4
_DC = _D // _NPH
_PK = _DC // 2
_VPT = _V // _NSUB


def _sc_body(x_hbm, cand_hbm, table_hbm, out_hbm, cand_v, x_v, rows_v, out_v,
             chunk_s, *sems):
    wid = lax.axis_index("s") * 2 + lax.axis_index("c")
    sid = lax.axis_index("s")
    base = wid * _RPW
    pltpu.sync_copy(x_hbm.at[pl.ds(base, _RPW)], x_v)
    pltpu.sync_copy(cand_hbm.at[pl.ds(base, _RPW)], cand_v)

    lane = lax.broadcasted_iota(jnp.int32, (16,), 0)
    himask = jnp.full((16,), -65536, jnp.int32)

    def gather_descs(row, b):
        return (
            pltpu.make_async_copy(chunk_s.at[cand_v.at[row, 0]],
                                  rows_v.at[b, pl.ds(0, _GH)], sems[b]),
            pltpu.make_async_copy(chunk_s.at[cand_v.at[row, 1]],
                                  rows_v.at[b, pl.ds(_GH, _GH)], sems[b]),
        )

    def issue(row, b):
        for desc in gather_descs(row, b):
            desc.start()

    for p in range(_NPH):
        pltpu.sync_copy(table_hbm.at[p, pl.ds(sid * _VPT, _VPT)],
                        chunk_s.at[pl.ds(sid * _VPT, _VPT)])
        plsc.subcore_barrier()

        def compute(row, b):
            bb = jnp.full((16,), b, jnp.int32)
            rr = jnp.full((16,), row, jnp.int32)

            def d_body(k, accs):
                kk = jnp.full((16,), k, jnp.int32)
                xb0 = plsc.load_gather(x_v, [rr, kk * 2 + (p * _DC)])
                xb1 = plsc.load_gather(x_v, [rr, kk * 2 + (p * _DC + 1)])
                new = []
                for j in range(_NCH):
                    w = plsc.load_gather(rows_v, [bb, lane + (16 * j), kk])
                    f0 = plsc.bitcast(w << 16, jnp.float32)
                    f1 = plsc.bitcast(w & himask, jnp.float32)
                    new.append(accs[j] + xb0 * f0 + xb1 * f1)
                return tuple(new)

            accs = lax.fori_loop(
                0, _PK, d_body,
                tuple(jnp.zeros((16,), jnp.float32) for _ in range(_NCH)))
            for j in range(_NCH):
                if p == 0:
                    out_v[row, pl.ds(16 * j, 16)] = accs[j]
                else:
                    out_v[row, pl.ds(16 * j, 16)] = (
                        out_v[row, pl.ds(16 * j, 16)] + accs[j])

        for b in range(_NBUF - 1):
            issue(b, b)

        def outer_body(r2, carry):
            for b in range(_NBUF):
                row = r2 * _NBUF + b
                nxt = row + (_NBUF - 1)

                @pl.when(nxt < _RPW)
                def _():
                    issue(nxt, (b + _NBUF - 1) % _NBUF)

                for desc in gather_descs(row, b):
                    desc.wait()
                compute(row, b)
            return carry

        lax.fori_loop(0, _RPW // _NBUF, outer_body, 0)
        plsc.subcore_barrier()

    pltpu.sync_copy(out_v, out_hbm.at[pl.ds(base, _RPW)])


def kernel(x, candidates, table):
    cand = candidates.astype(jnp.int32).reshape(_B, 2, _GH)

    tu = jax.lax.bitcast_convert_type(
        table.astype(jnp.bfloat16), jnp.uint16)
    w = tu[:, 0::2].astype(jnp.uint32) | (
        tu[:, 1::2].astype(jnp.uint32) << 16)
    table_t = jax.lax.bitcast_convert_type(
        w, jnp.int32).reshape(_V, _NPH, _PK).transpose(1, 0, 2)

    mesh = plsc.VectorSubcoreMesh(core_axis_name="c", subcore_axis_name="s")
    out = pl.kernel(
        _sc_body,
        mesh=mesh,
        compiler_params=pltpu.CompilerParams(
            needs_layout_passes=False, use_tc_tiling_on_sc=False),
        out_type=jax.ShapeDtypeStruct((_B, _CP), jnp.float32),
        scratch_types=[
            pltpu.VMEM((_RPW, 2, _GH), jnp.int32),
            pltpu.VMEM((_RPW, _D), jnp.float32),
            pltpu.VMEM((_NBUF, _CP, _PK), jnp.int32),
            pltpu.VMEM((_RPW, _CP), jnp.float32),
            pltpu.MemorySpace.VMEM_SHARED((_V, _PK), jnp.int32),
        ] + [pltpu.SemaphoreType.DMA] * _NBUF,
    )(x, cand, table_t)
    return out[:, :_C]
